# baseline (speedup 1.0000x reference)
import functools

import jax
import jax.numpy as jnp
from jax import lax
from jax.experimental import pallas as pl
from jax.experimental.pallas import tpu as pltpu
from jax.experimental.pallas import tpu_sc as plsc

VOCAB = 1000000
EMBED_DIM = 64
BATCH = 4096
SEQ_LEN = 200

NUM_CORES = 2
NUM_SUBCORES = 16
NW = NUM_CORES * NUM_SUBCORES
TOTAL = BATCH * SEQ_LEN
B_PER_W = TOTAL // NW
CHUNK = 128
NCHUNK = B_PER_W // CHUNK
NBUF = 4
NGROUP = NCHUNK // NBUF


@functools.partial(jax.jit, static_argnums=())
def _embed(idx3, table):
    mesh = plsc.VectorSubcoreMesh(
        core_axis_name="c", subcore_axis_name="s",
        num_cores=NUM_CORES, num_subcores=NUM_SUBCORES)

    @functools.partial(
        pl.kernel,
        mesh=mesh,
        out_type=jax.ShapeDtypeStruct((TOTAL, EMBED_DIM), jnp.float32),
        scratch_types=[
            pltpu.VMEM((NCHUNK, CHUNK), jnp.int32),
            pltpu.VMEM((NBUF, CHUNK, EMBED_DIM), jnp.float32),
            pltpu.SemaphoreType.DMA((NBUF,)),
            pltpu.SemaphoreType.DMA((NBUF,)),
        ],
        compiler_params=pltpu.CompilerParams(use_tc_tiling_on_sc=False),
    )
    def emb_kernel(idx_hbm, table_hbm, out_hbm, idx_v, rows_v, gsem, wsem):
        wid = lax.axis_index("s") * NUM_CORES + lax.axis_index("c")
        base = wid * B_PER_W
        pltpu.sync_copy(idx_hbm.at[wid], idx_v)

        def gather_start(j, b):
            pltpu.async_copy(table_hbm.at[idx_v.at[j]], rows_v.at[b],
                             gsem.at[b])

        def gather_wait(j, b):
            pltpu.make_async_copy(table_hbm.at[idx_v.at[j]], rows_v.at[b],
                                  gsem.at[b]).wait()

        def write_start(j, b):
            pltpu.async_copy(rows_v.at[b],
                             out_hbm.at[pl.ds(base + j * CHUNK, CHUNK)],
                             wsem.at[b])

        def write_wait(j, b):
            pltpu.make_async_copy(rows_v.at[b],
                                  out_hbm.at[pl.ds(base + j * CHUNK, CHUNK)],
                                  wsem.at[b]).wait()

        for b in range(NBUF):
            gather_start(b, b)

        def body(g, carry):
            j0 = g * NBUF
            for b in range(NBUF):
                gather_wait(j0 + b, b)
                write_start(j0 + b, b)
            for b in range(NBUF):
                write_wait(j0 + b, b)

                @pl.when(g < NGROUP - 1)
                def _():
                    gather_start(j0 + NBUF + b, b)
            return carry

        lax.fori_loop(0, NGROUP, body, 0)

    return emb_kernel(idx3, table)


def kernel(text, table):
    idx3 = text.reshape(NW, NCHUNK, CHUNK)
    out = _embed(idx3, table)
    return out.reshape(BATCH, SEQ_LEN, EMBED_DIM)
